# baseline (speedup 1.0000x reference)
import jax
import jax.numpy as jnp
from jax import lax
from jax.experimental import pallas as pl
from jax.experimental.pallas import tpu as pltpu
from jax.experimental.pallas import tpu_sc as plsc

def _body(x_hbm, y_hbm, o_hbm, oacc, sem):
    wid = lax.axis_index("s") * 2 + lax.axis_index("c")
    oacc[...] = jnp.zeros((16,), jnp.float32)
    pltpu.sync_copy(oacc, o_hbm.at[wid])

_call = pl.kernel(
    _body,
    out_type=jax.ShapeDtypeStruct((32, 16), jnp.float32),
    mesh=plsc.VectorSubcoreMesh(core_axis_name="c", subcore_axis_name="s"),
    scratch_types=[
        pltpu.VMEM((16,), jnp.float32),
        pltpu.SemaphoreType.DMA,
    ],
    compiler_params=pltpu.CompilerParams(
        use_tc_tiling_on_sc=False, needs_layout_passes=False),
)

def kernel(x, y):
    return jnp.sum(_call(x, y))


# speedup vs baseline: 21.6633x; 21.6633x over previous
import jax
import jax.numpy as jnp
from jax import lax
from jax.experimental import pallas as pl
from jax.experimental.pallas import tpu as pltpu

_CH = 25
_B = 64
_C = 10647
_BC = 1536
_NCB = 7


def _tc_body(ym_ref, x_ref, y_ref, o_ref):
    cb = pl.program_id(0)
    ch = pl.program_id(1)

    @pl.when((cb == 0) & (ch == 0))
    def _():
        o_ref[0] = 0.0

    xb = x_ref[0]
    yb = y_ref[0]
    ym = ym_ref[0]
    col = cb * _BC + lax.broadcasted_iota(jnp.int32, (_B, _BC), 1)
    valid = col < _C
    mb = (ym > 0.5) & valid
    d = yb - xb
    sq = d * d
    t = jnp.where(mb, sq, 0.0)
    chfac = jnp.where(ch == 0, jnp.float32(0.1), jnp.float32(0.0))
    bg = jnp.where(mb, 0.0, jnp.where(valid, xb * xb, 0.0)) * chfac
    o_ref[0] += jnp.sum(t + bg)


_tc_call = pl.pallas_call(
    _tc_body,
    grid=(_NCB, _CH),
    in_specs=[
        pl.BlockSpec((1, _B, _BC), lambda cb, ch: (0, 0, cb)),
        pl.BlockSpec((1, _B, _BC), lambda cb, ch: (ch, 0, cb)),
        pl.BlockSpec((1, _B, _BC), lambda cb, ch: (ch, 0, cb)),
    ],
    out_specs=pl.BlockSpec(memory_space=pltpu.SMEM),
    out_shape=jax.ShapeDtypeStruct((1,), jnp.float32),
)


def kernel(x, y):
    xt = jnp.transpose(x, (2, 0, 1))
    yt = jnp.transpose(y, (2, 0, 1))
    out = _tc_call(yt, xt, yt)
    return out[0]


# speedup vs baseline: 62.2282x; 2.8725x over previous
import jax
import jax.numpy as jnp
from jax import lax
from jax.experimental import pallas as pl
from jax.experimental.pallas import tpu as pltpu

_CH = 25
_B = 64
_C = 10647


def _fold8(t):
    parts = [t[i * 8:(i + 1) * 8] for i in range(8)]
    while len(parts) > 1:
        parts = [a + b for a, b in zip(parts[::2], parts[1::2])]
    return parts[0]


def _tc_body(ym_ref, x_ref, y_ref, o_ref, acc_ref):
    ch = pl.program_id(0)

    xb = x_ref[0]
    yb = y_ref[0]
    ym = ym_ref[0]
    mb = ym > 0.5
    d = yb - xb
    sq = d * d

    @pl.when(ch == 0)
    def _():
        acc_ref[...] = _fold8(jnp.where(mb, sq, 0.1 * (xb * xb)))

    @pl.when(ch > 0)
    def _():
        acc_ref[...] += _fold8(jnp.where(mb, sq, 0.0))

    @pl.when(ch == _CH - 1)
    def _():
        o_ref[0] = jnp.sum(acc_ref[...])


_tc_call = pl.pallas_call(
    _tc_body,
    grid=(_CH,),
    in_specs=[
        pl.BlockSpec((1, _B, _C), lambda ch: (0, 0, 0)),
        pl.BlockSpec((1, _B, _C), lambda ch: (ch, 0, 0)),
        pl.BlockSpec((1, _B, _C), lambda ch: (ch, 0, 0)),
    ],
    out_specs=pl.BlockSpec(memory_space=pltpu.SMEM),
    out_shape=jax.ShapeDtypeStruct((1,), jnp.float32),
    scratch_shapes=[pltpu.VMEM((8, _C), jnp.float32)],
)


def kernel(x, y):
    xt = jnp.transpose(x, (2, 0, 1))
    yt = jnp.transpose(y, (2, 0, 1))
    out = _tc_call(yt, xt, yt)
    return out[0]
